# baseline (speedup 1.0000x reference)
import functools

import jax
import jax.numpy as jnp
from jax import lax
from jax.experimental import pallas as pl
from jax.experimental.pallas import tpu as pltpu
from jax.experimental.pallas import tpu_sc as plsc

BATCH = 16384
N_FIELDS = 26
EMBED_DIM = 32

NUM_CORES = 2
NUM_SUBCORES = 16
NW = NUM_CORES * NUM_SUBCORES

G = 128
N_BLOCKS = N_FIELDS * (BATCH // G)
BPW = N_BLOCKS // NW

_mesh = plsc.VectorSubcoreMesh(core_axis_name="c", subcore_axis_name="s")


@functools.partial(
    pl.kernel,
    out_type=jax.ShapeDtypeStruct((N_FIELDS, 4, BATCH // G, 8, G), jnp.float32),
    mesh=_mesh,
    scratch_types=[
        pltpu.VMEM((BPW, G), jnp.int32),
        pltpu.VMEM((G, EMBED_DIM), jnp.float32),
        pltpu.VMEM((G, EMBED_DIM), jnp.float32),
        pltpu.VMEM((4, 8, G), jnp.float32),
        pltpu.VMEM((4, 8, G), jnp.float32),
        pltpu.SemaphoreType.DMA,
        pltpu.SemaphoreType.DMA,
        pltpu.SemaphoreType.DMA,
        pltpu.SemaphoreType.DMA,
    ],
    compiler_params=pltpu.CompilerParams(
        use_tc_tiling_on_sc=False, needs_layout_passes=False
    ),
)
def _emb_lookup(idx_hbm, table_hbm, out_hbm, idx_v, rows0, rows1, blk0, blk1,
                sem_g0, sem_g1, sem_o0, sem_o1):
    w = lax.axis_index("s") * NUM_CORES + lax.axis_index("c")
    k0 = w * BPW
    rows = [rows0, rows1]
    blks = [blk0, blk1]
    sem_g = [sem_g0, sem_g1]
    sem_o = [sem_o0, sem_o1]

    pltpu.sync_copy(idx_hbm.at[pl.ds(k0, BPW)], idx_v)

    lane = lax.iota(jnp.int32, 16)
    rvecs = [lane + 16 * g for g in range(8)]

    def fire_gather(i, b):
        pltpu.async_copy(table_hbm.at[idx_v.at[i]], rows[b], sem_g[b])

    def wait_gather(b):
        pltpu.make_async_copy(table_hbm.at[pl.ds(0, G)], rows[b], sem_g[b]).wait()

    def fire_out(i, b):
        kg = k0 + i
        f = lax.shift_right_logical(kg, 7)
        tc = lax.bitwise_and(kg, 127)
        for tr in range(4):
            pltpu.async_copy(blks[b].at[tr], out_hbm.at[f, tr, tc], sem_o[b])

    def wait_out(b):
        for tr in range(4):
            pltpu.make_async_copy(blks[b].at[tr], out_hbm.at[0, tr, 0],
                                  sem_o[b]).wait()

    def extract(b):
        src = rows[b]
        dst = blks[b]
        for tr in range(4):
            for ri in range(8):
                c = jnp.full((16,), tr * 8 + ri, jnp.int32)
                for g in range(8):
                    vals = plsc.load_gather(src, [rvecs[g], c])
                    dst[tr, ri, pl.ds(16 * g, 16)] = vals

    fire_gather(0, 0)

    @pl.loop(0, BPW, step=2)
    def _outer(i0):
        for b in range(2):
            i = i0 + b
            @pl.when(i < BPW - 1)
            def _():
                fire_gather(i + 1, 1 - b)
            wait_gather(b)
            @pl.when(i >= 2)
            def _():
                wait_out(b)
            extract(b)
            fire_out(i, b)

    wait_out(0)
    wait_out(1)


def kernel(x, embedding_weight):
    idx = x.T.reshape(N_BLOCKS, G).astype(jnp.int32)
    out5 = _emb_lookup(idx, embedding_weight)
    return out5.transpose(2, 4, 0, 1, 3).reshape(BATCH, N_FIELDS, EMBED_DIM)


# speedup vs baseline: 1.4074x; 1.4074x over previous
import functools

import jax
import jax.numpy as jnp
from jax import lax
from jax.experimental import pallas as pl
from jax.experimental.pallas import tpu as pltpu
from jax.experimental.pallas import tpu_sc as plsc

BATCH = 16384
N_FIELDS = 26
EMBED_DIM = 32

NUM_CORES = 2
NUM_SUBCORES = 16
NW = NUM_CORES * NUM_SUBCORES

G = 128
N_BLOCKS = N_FIELDS * (BATCH // G)
BPW = N_BLOCKS // NW

_mesh = plsc.VectorSubcoreMesh(core_axis_name="c", subcore_axis_name="s")


@functools.partial(
    pl.kernel,
    out_type=jax.ShapeDtypeStruct((N_FIELDS, 4, BATCH // G, 8, G), jnp.float32),
    mesh=_mesh,
    scratch_types=[
        pltpu.VMEM((BPW, G), jnp.int32),
        pltpu.VMEM((G, EMBED_DIM), jnp.float32),
        pltpu.VMEM((G, EMBED_DIM), jnp.float32),
        pltpu.VMEM((EMBED_DIM, 133), jnp.float32),
        pltpu.VMEM((EMBED_DIM, 133), jnp.float32),
        pltpu.SemaphoreType.DMA,
        pltpu.SemaphoreType.DMA,
        pltpu.SemaphoreType.DMA,
        pltpu.SemaphoreType.DMA,
    ],
    compiler_params=pltpu.CompilerParams(
        use_tc_tiling_on_sc=False, needs_layout_passes=False
    ),
)
def _emb_lookup(idx_hbm, table_hbm, out_hbm, idx_v, rows0, rows1, blk0, blk1,
                sem_g0, sem_g1, sem_o0, sem_o1):
    w = lax.axis_index("s") * NUM_CORES + lax.axis_index("c")
    k0 = w * BPW
    rows = [rows0, rows1]
    blks = [blk0, blk1]
    sem_g = [sem_g0, sem_g1]
    sem_o = [sem_o0, sem_o1]

    pltpu.sync_copy(idx_hbm.at[pl.ds(k0, BPW)], idx_v)

    lane = lax.iota(jnp.int32, 16)
    cvecs = [lane + 16 * h for h in range(2)]

    def fire_gather(i, b):
        pltpu.async_copy(table_hbm.at[idx_v.at[i]], rows[b], sem_g[b])

    def wait_gather(b):
        pltpu.make_async_copy(table_hbm.at[pl.ds(0, G)], rows[b], sem_g[b]).wait()

    def fire_out(i, b):
        kg = k0 + i
        f = lax.shift_right_logical(kg, 7)
        tc = lax.bitwise_and(kg, 127)
        for tr in range(4):
            pltpu.async_copy(blks[b].at[pl.ds(tr * 8, 8), pl.ds(0, G)],
                             out_hbm.at[f, tr, tc], sem_o[b])

    def wait_out(b):
        for tr in range(4):
            pltpu.make_async_copy(blks[b].at[pl.ds(tr * 8, 8), pl.ds(0, G)],
                                  out_hbm.at[0, tr, 0], sem_o[b]).wait()

    def extract(b):
        src = rows[b]
        dst = blks[b]
        for j in range(G):
            jv = jnp.full((16,), j, jnp.int32)
            for h in range(2):
                vals = src[j, pl.ds(16 * h, 16)]
                plsc.store_scatter(dst, [cvecs[h], jv], vals)

    fire_gather(0, 0)

    @pl.loop(0, BPW, step=2)
    def _outer(i0):
        for b in range(2):
            i = i0 + b
            @pl.when(i < BPW - 1)
            def _():
                fire_gather(i + 1, 1 - b)
            wait_gather(b)
            @pl.when(i >= 2)
            def _():
                wait_out(b)
            extract(b)
            fire_out(i, b)

    wait_out(0)
    wait_out(1)


def kernel(x, embedding_weight):
    idx = x.T.reshape(N_BLOCKS, G).astype(jnp.int32)
    out5 = _emb_lookup(idx, embedding_weight)
    return out5.transpose(2, 4, 0, 1, 3).reshape(BATCH, N_FIELDS, EMBED_DIM)


# speedup vs baseline: 1.5716x; 1.1166x over previous
import functools

import jax
import jax.numpy as jnp
from jax import lax
from jax.experimental import pallas as pl
from jax.experimental.pallas import tpu as pltpu
from jax.experimental.pallas import tpu_sc as plsc

BATCH = 16384
N_FIELDS = 26
EMBED_DIM = 32
VOCAB = 1000000

NUM_CORES = 2
NUM_SUBCORES = 16
NW = NUM_CORES * NUM_SUBCORES

G = 128
N_BLOCKS = N_FIELDS * (BATCH // G)
BPW = N_BLOCKS // NW

N_TCOL = 7813
N_TCOL_FULL = 7812
COLS_PER_W = 244
AK = 2
ABATCH = COLS_PER_W // AK
VOCAB_PAD = N_TCOL * 128
PACK_ROWS = VOCAB_PAD * EMBED_DIM // 128

_mesh = plsc.VectorSubcoreMesh(core_axis_name="c", subcore_axis_name="s")


@functools.partial(
    pl.kernel,
    out_type=jax.ShapeDtypeStruct((PACK_ROWS, 128), jnp.float32),
    mesh=_mesh,
    scratch_types=[
        pltpu.VMEM((AK * 32, 128), jnp.float32),
        pltpu.VMEM((AK * 32, 128), jnp.float32),
        pltpu.VMEM((AK * 32, 128), jnp.float32),
        pltpu.VMEM((AK * 32, 128), jnp.float32),
        pltpu.VMEM((32, 64), jnp.float32),
        pltpu.VMEM((4352,), jnp.float32),
        pltpu.SemaphoreType.DMA,
        pltpu.SemaphoreType.DMA,
        pltpu.SemaphoreType.DMA,
        pltpu.SemaphoreType.DMA,
    ],
    compiler_params=pltpu.CompilerParams(
        use_tc_tiling_on_sc=True, needs_layout_passes=False
    ),
)
def _detile(tableT_hbm, out_hbm, s0, s1, e0, e1, s64, dpad,
            si0, si1, so0, so1):
    w = lax.axis_index("s") * NUM_CORES + lax.axis_index("c")
    c0 = w * COLS_PER_W
    ss = [s0, s1]
    es = [e0, e1]
    si = [si0, si1]
    so = [so0, so1]

    lane = lax.iota(jnp.int32, 16)
    basev = [(lane + 16 * g) * 33 for g in range(8)]

    def fire_in(b, p):
        for k in range(AK):
            col = c0 + b * AK + k
            pltpu.async_copy(tableT_hbm.at[:, pl.ds(128 * col, 128)],
                             ss[p].at[pl.ds(32 * k, 32)], si[p])

    def wait_in(p):
        for k in range(AK):
            pltpu.make_async_copy(tableT_hbm.at[:, pl.ds(0, 128)],
                                  ss[p].at[pl.ds(32 * k, 32)], si[p]).wait()

    def fire_out(b, p):
        pltpu.async_copy(es[p], out_hbm.at[pl.ds((c0 + b * AK) * 32, AK * 32)],
                         so[p])

    def wait_out(p):
        pltpu.make_async_copy(es[p], out_hbm.at[pl.ds(0, AK * 32)],
                              so[p]).wait()

    def transpose_col(src, dst, nci):
        for c in range(32):
            for gidx in range(nci // 16):
                vals = src[c, pl.ds(16 * gidx, 16)]
                plsc.store_scatter(dpad, [basev[gidx] + c], vals)
        for a in range(nci * 32 // 128):
            for h in range(8):
                off = (4 * a + h // 2) * 33 + 16 * (h % 2)
                dst[a, pl.ds(16 * h, 16)] = dpad[pl.ds(off, 16)]

    def compute(p):
        for k in range(AK):
            transpose_col(ss[p].at[pl.ds(32 * k, 32)],
                          es[p].at[pl.ds(32 * k, 32)], 128)

    fire_in(0, 0)
    fire_in(1, 1)

    @pl.loop(0, ABATCH, step=2)
    def _outer(b0):
        for p in range(2):
            b = b0 + p
            wait_in(p)
            @pl.when(b >= 2)
            def _():
                wait_out(p)
            compute(p)
            fire_out(b, p)
            @pl.when(b + 2 < ABATCH)
            def _():
                fire_in(b + 2, p)

    wait_out(0)
    wait_out(1)

    @pl.when(w < 4)
    def _():
        cc = N_TCOL_FULL - 4 + w
        pltpu.sync_copy(tableT_hbm.at[:, pl.ds(128 * cc, 128)],
                        s0.at[pl.ds(0, 32)])
        transpose_col(s0.at[pl.ds(0, 32)], e0.at[pl.ds(0, 32)], 128)
        pltpu.sync_copy(e0.at[pl.ds(0, 32)], out_hbm.at[pl.ds(32 * cc, 32)])

    @pl.when(w == 30)
    def _():
        pltpu.sync_copy(tableT_hbm.at[:, pl.ds(128 * N_TCOL_FULL, 64)], s64)
        transpose_col(s64, e1.at[pl.ds(0, 16)], 64)
        pltpu.sync_copy(e1.at[pl.ds(0, 16)],
                        out_hbm.at[pl.ds(32 * N_TCOL_FULL, 16)])


@functools.partial(
    pl.kernel,
    out_type=jax.ShapeDtypeStruct((N_FIELDS, 4, BATCH // G, 8, G), jnp.float32),
    mesh=_mesh,
    scratch_types=[
        pltpu.VMEM((BPW, G), jnp.int32),
        pltpu.VMEM((G, EMBED_DIM), jnp.float32),
        pltpu.VMEM((G, EMBED_DIM), jnp.float32),
        pltpu.VMEM((EMBED_DIM, 133), jnp.float32),
        pltpu.VMEM((EMBED_DIM, 133), jnp.float32),
        pltpu.SemaphoreType.DMA,
        pltpu.SemaphoreType.DMA,
        pltpu.SemaphoreType.DMA,
        pltpu.SemaphoreType.DMA,
    ],
    compiler_params=pltpu.CompilerParams(
        use_tc_tiling_on_sc=False, needs_layout_passes=False
    ),
)
def _emb_lookup(idx_hbm, table_hbm, out_hbm, idx_v, rows0, rows1, blk0, blk1,
                sem_g0, sem_g1, sem_o0, sem_o1):
    w = lax.axis_index("s") * NUM_CORES + lax.axis_index("c")
    k0 = w * BPW
    rows = [rows0, rows1]
    blks = [blk0, blk1]
    sem_g = [sem_g0, sem_g1]
    sem_o = [sem_o0, sem_o1]

    pltpu.sync_copy(idx_hbm.at[pl.ds(k0, BPW)], idx_v)

    lane = lax.iota(jnp.int32, 16)
    cvecs = [lane + 16 * h for h in range(2)]

    def fire_gather(i, b):
        pltpu.async_copy(table_hbm.at[idx_v.at[i]], rows[b], sem_g[b])

    def wait_gather(b):
        pltpu.make_async_copy(table_hbm.at[pl.ds(0, G)], rows[b], sem_g[b]).wait()

    def fire_out(i, b):
        kg = k0 + i
        f = lax.shift_right_logical(kg, 7)
        tc = lax.bitwise_and(kg, 127)
        for tr in range(4):
            pltpu.async_copy(blks[b].at[pl.ds(tr * 8, 8), pl.ds(0, G)],
                             out_hbm.at[f, tr, tc], sem_o[b])

    def wait_out(b):
        for tr in range(4):
            pltpu.make_async_copy(blks[b].at[pl.ds(tr * 8, 8), pl.ds(0, G)],
                                  out_hbm.at[0, tr, 0], sem_o[b]).wait()

    def extract(b):
        src = rows[b]
        dst = blks[b]
        for j in range(G):
            jv = jnp.full((16,), j, jnp.int32)
            for h in range(2):
                vals = src[j, pl.ds(16 * h, 16)]
                plsc.store_scatter(dst, [cvecs[h], jv], vals)

    fire_gather(0, 0)

    @pl.loop(0, BPW, step=2)
    def _outer(i0):
        for b in range(2):
            i = i0 + b
            @pl.when(i < BPW - 1)
            def _():
                fire_gather(i + 1, 1 - b)
            wait_gather(b)
            @pl.when(i >= 2)
            def _():
                wait_out(b)
            extract(b)
            fire_out(i, b)

    wait_out(0)
    wait_out(1)


def kernel(x, embedding_weight):
    idx = x.T.reshape(N_BLOCKS, G).astype(jnp.int32)
    table_packed = _detile(embedding_weight.T)
    table_rm = table_packed.reshape(VOCAB_PAD, EMBED_DIM)
    out5 = _emb_lookup(idx, table_rm)
    return out5.transpose(2, 4, 0, 1, 3).reshape(BATCH, N_FIELDS, EMBED_DIM)


# speedup vs baseline: 1.8335x; 1.1667x over previous
import functools

import jax
import jax.numpy as jnp
from jax import lax
from jax.experimental import pallas as pl
from jax.experimental.pallas import tpu as pltpu
from jax.experimental.pallas import tpu_sc as plsc

BATCH = 16384
N_FIELDS = 26
EMBED_DIM = 32
VOCAB = 1000000

NUM_CORES = 2
NUM_SUBCORES = 16
NW = NUM_CORES * NUM_SUBCORES

G = 128
N_BLOCKS = N_FIELDS * (BATCH // G)
BPW = N_BLOCKS // NW

N_TCOL = 7813
N_TCOL_FULL = 7812
COLS_PER_W = 244
AK = 2
ABATCH = COLS_PER_W // AK
VOCAB_PAD = N_TCOL * 128
PACK_ROWS = VOCAB_PAD * EMBED_DIM // 128

_mesh = plsc.VectorSubcoreMesh(core_axis_name="c", subcore_axis_name="s")


ROW_PITCH = 32
COL_WORDS = 128 * ROW_PITCH
INTER_WORDS = N_TCOL * COL_WORDS


@functools.partial(
    pl.kernel,
    out_type=jax.ShapeDtypeStruct((INTER_WORDS,), jnp.float32),
    mesh=_mesh,
    scratch_types=[
        pltpu.VMEM((AK * 32, 128), jnp.float32),
        pltpu.VMEM((AK * 32, 128), jnp.float32),
        pltpu.VMEM((32, 64), jnp.float32),
        pltpu.VMEM((COL_WORDS,), jnp.float32),
        pltpu.VMEM((COL_WORDS,), jnp.float32),
        pltpu.SemaphoreType.DMA,
        pltpu.SemaphoreType.DMA,
        pltpu.SemaphoreType.DMA,
        pltpu.SemaphoreType.DMA,
    ],
    compiler_params=pltpu.CompilerParams(
        use_tc_tiling_on_sc=True, needs_layout_passes=False
    ),
)
def _detile(tableT_hbm, out_hbm, s0, s1, s64, dpad0, dpad1,
            si0, si1, so0, so1):
    w = lax.axis_index("s") * NUM_CORES + lax.axis_index("c")
    c0 = w * COLS_PER_W
    ss = [s0, s1]
    dps = [dpad0, dpad1]
    si = [si0, si1]
    so = [so0, so1]

    lane = lax.iota(jnp.int32, 16)
    civ = [lane + 16 * g for g in range(8)]
    base32 = [(lane + 16 * g) * 32 for g in range(8)]

    def fire_in(b, p):
        for k in range(AK):
            col = c0 + b * AK + k
            pltpu.async_copy(tableT_hbm.at[:, pl.ds(128 * col, 128)],
                             ss[p].at[pl.ds(32 * k, 32)], si[p])

    def wait_in(p):
        for k in range(AK):
            pltpu.make_async_copy(tableT_hbm.at[:, pl.ds(0, 128)],
                                  ss[p].at[pl.ds(32 * k, 32)], si[p]).wait()

    def fire_out(col, k):
        pltpu.async_copy(dps[k], out_hbm.at[pl.ds(col * COL_WORDS, COL_WORDS)],
                         so[k])

    def wait_out(k):
        pltpu.make_async_copy(dps[k], out_hbm.at[pl.ds(0, COL_WORDS)],
                              so[k]).wait()

    def scatter_col(src, nci, dp):
        for c in range(32):
            for gidx in range(nci // 16):
                vals = src[c, pl.ds(16 * gidx, 16)]
                idxv = base32[gidx] + ((civ[gidx] + c) & 31)
                plsc.store_scatter(dp, [idxv], vals)

    fire_in(0, 0)
    fire_in(1, 1)

    @pl.loop(0, ABATCH, step=2)
    def _outer(b0):
        for p in range(2):
            b = b0 + p
            wait_in(p)
            for k in range(AK):
                @pl.when(b >= 1)
                def _():
                    wait_out(k)
                scatter_col(ss[p].at[pl.ds(32 * k, 32)], 128, dps[k])
                fire_out(c0 + b * AK + k, k)
            @pl.when(b + 2 < ABATCH)
            def _():
                fire_in(b + 2, p)

    wait_out(0)
    wait_out(1)

    @pl.when(w < 4)
    def _():
        cc = N_TCOL_FULL - 4 + w
        pltpu.sync_copy(tableT_hbm.at[:, pl.ds(128 * cc, 128)],
                        s0.at[pl.ds(0, 32)])
        scatter_col(s0.at[pl.ds(0, 32)], 128, dpad0)
        pltpu.sync_copy(dpad0, out_hbm.at[pl.ds(cc * COL_WORDS, COL_WORDS)])

    @pl.when(w == 30)
    def _():
        pltpu.sync_copy(tableT_hbm.at[:, pl.ds(128 * N_TCOL_FULL, 64)], s64)
        scatter_col(s64, 64, dpad1)
        pltpu.sync_copy(dpad1.at[pl.ds(0, 64 * ROW_PITCH)],
                        out_hbm.at[pl.ds(N_TCOL_FULL * COL_WORDS,
                                         64 * ROW_PITCH)])


@functools.partial(
    pl.kernel,
    out_type=jax.ShapeDtypeStruct((N_FIELDS, 4, BATCH // G, 8, G), jnp.float32),
    mesh=_mesh,
    scratch_types=[
        pltpu.VMEM((BPW, G), jnp.int32),
        pltpu.VMEM((G, ROW_PITCH), jnp.float32),
        pltpu.VMEM((G, ROW_PITCH), jnp.float32),
        pltpu.VMEM((EMBED_DIM, 133), jnp.float32),
        pltpu.VMEM((EMBED_DIM, 133), jnp.float32),
        pltpu.SemaphoreType.DMA,
        pltpu.SemaphoreType.DMA,
        pltpu.SemaphoreType.DMA,
        pltpu.SemaphoreType.DMA,
    ],
    compiler_params=pltpu.CompilerParams(
        use_tc_tiling_on_sc=False, needs_layout_passes=False
    ),
)
def _emb_lookup(idx_hbm, table_hbm, out_hbm, idx_v, rows0, rows1, blk0, blk1,
                sem_g0, sem_g1, sem_o0, sem_o1):
    w = lax.axis_index("s") * NUM_CORES + lax.axis_index("c")
    k0 = w * BPW
    rows = [rows0, rows1]
    blks = [blk0, blk1]
    sem_g = [sem_g0, sem_g1]
    sem_o = [sem_o0, sem_o1]

    pltpu.sync_copy(idx_hbm.at[pl.ds(k0, BPW)], idx_v)

    lane = lax.iota(jnp.int32, 16)
    cvecs = [lane + 16 * h for h in range(2)]

    def fire_gather(i, b):
        pltpu.async_copy(table_hbm.at[idx_v.at[i]], rows[b], sem_g[b])

    def wait_gather(b):
        pltpu.make_async_copy(table_hbm.at[pl.ds(0, G)], rows[b], sem_g[b]).wait()

    def fire_out(i, b):
        kg = k0 + i
        f = lax.shift_right_logical(kg, 7)
        tc = lax.bitwise_and(kg, 127)
        for tr in range(4):
            pltpu.async_copy(blks[b].at[pl.ds(tr * 8, 8), pl.ds(0, G)],
                             out_hbm.at[f, tr, tc], sem_o[b])

    def wait_out(b):
        for tr in range(4):
            pltpu.make_async_copy(blks[b].at[pl.ds(tr * 8, 8), pl.ds(0, G)],
                                  out_hbm.at[0, tr, 0], sem_o[b]).wait()

    def extract(b):
        src = rows[b]
        dst = blks[b]
        for j in range(G):
            jv = jnp.full((16,), j, jnp.int32)
            for h in range(2):
                vals = src[j, pl.ds(16 * h, 16)]
                cv = (cvecs[h] + (32 - (j & 31))) & 31
                plsc.store_scatter(dst, [cv, jv], vals)

    fire_gather(0, 0)

    @pl.loop(0, BPW, step=2)
    def _outer(i0):
        for b in range(2):
            i = i0 + b
            @pl.when(i < BPW - 1)
            def _():
                fire_gather(i + 1, 1 - b)
            wait_gather(b)
            @pl.when(i >= 2)
            def _():
                wait_out(b)
            extract(b)
            fire_out(i, b)

    wait_out(0)
    wait_out(1)


def kernel(x, embedding_weight):
    idx = x.T.reshape(N_BLOCKS, G).astype(jnp.int32)
    table_packed = _detile(embedding_weight.T)
    table_rm = table_packed.reshape(VOCAB_PAD, ROW_PITCH)
    out5 = _emb_lookup(idx, table_rm)
    return out5.transpose(2, 4, 0, 1, 3).reshape(BATCH, N_FIELDS, EMBED_DIM)
